# baseline (speedup 1.0000x reference)
import functools

import jax
import jax.numpy as jnp
from jax import lax
from jax.experimental import pallas as pl
from jax.experimental.pallas import tpu as pltpu
from jax.experimental.pallas import tpu_sc as plsc

_CUTOFF = 5.0
_NUM_BASIS = 16
_EPG = 8
_ROW = 3 * _EPG
_BR = 2000

_NC = 2
_NS = 16
_NW = _NC * _NS


def _bessel_body(e_ref, o_ref):
    e = e_ref[...]
    s = e * e
    jj = lax.broadcasted_iota(jnp.int32, (_ROW, 128), 0) // 3
    ll = lax.broadcasted_iota(jnp.int32, (_ROW, 128), 1) // _NUM_BASIS
    proj = (jj == ll).astype(jnp.float32)
    r2 = lax.dot_general(s, proj, (((1,), (0,)), ((), ())),
                         preferred_element_type=jnp.float32)
    r = jnp.sqrt(r2)
    n = (lax.broadcasted_iota(jnp.int32, r.shape, 1) % _NUM_BASIS + 1
         ).astype(jnp.float32)
    arg = (jnp.pi / _CUTOFF) * n * r
    o_ref[...] = jnp.sqrt(2.0 / _CUTOFF) * jnp.sin(arg) / r


def _bessel_edges(edge_attr):
    ne = edge_attr.shape[0]
    rows = ne // _EPG
    e = edge_attr.reshape(rows, _ROW)
    grid = rows // _BR
    out = pl.pallas_call(
        _bessel_body,
        grid=(grid,),
        in_specs=[pl.BlockSpec((_BR, _ROW), lambda i: (i, 0))],
        out_specs=pl.BlockSpec((_BR, 128), lambda i: (i, 0)),
        out_shape=jax.ShapeDtypeStruct((rows, 128), jnp.float32),
    )(e)
    return out.reshape(ne, _NUM_BASIS)


def _make_sc_gather(n_pad):
    b_per_w = n_pad // _NW
    mesh = plsc.VectorSubcoreMesh(core_axis_name="c", subcore_axis_name="s")

    @functools.partial(
        pl.kernel,
        mesh=mesh,
        compiler_params=pltpu.CompilerParams(use_tc_tiling_on_sc=False),
        out_type=jax.ShapeDtypeStruct((n_pad, _NUM_BASIS), jnp.float32),
        scratch_types=[
            pltpu.VMEM((b_per_w,), jnp.int32),
            pltpu.VMEM((b_per_w, _NUM_BASIS), jnp.float32),
            pltpu.SemaphoreType.DMA,
        ],
    )
    def gather(table_hbm, idx_hbm, out_hbm, idx_v, rows_v, sem):
        wid = lax.axis_index("s") * _NC + lax.axis_index("c")
        base = wid * b_per_w
        pltpu.sync_copy(idx_hbm.at[pl.ds(base, b_per_w)], idx_v)
        pltpu.async_copy(table_hbm.at[idx_v], rows_v, sem).wait()
        pltpu.sync_copy(rows_v, out_hbm.at[pl.ds(base, b_per_w)])

    return gather


def kernel(x, edge_attr, W_x, W_z):
    n = x.shape[0]
    table = jnp.concatenate([W_x, W_z], axis=1)
    n_pad = ((n + 8 * _NW - 1) // (8 * _NW)) * (8 * _NW)
    xi = jnp.pad(x.astype(jnp.int32), (0, n_pad - n))
    nodes = _make_sc_gather(n_pad)(table, xi)
    h_node_x = nodes[:n, : W_x.shape[1]]
    h_node_z = nodes[:n, W_x.shape[1]:]
    h_edge = _bessel_edges(edge_attr)
    return h_node_x, h_node_z, h_edge


# speedup vs baseline: 3.7144x; 3.7144x over previous
import functools

import jax
import jax.numpy as jnp
from jax import lax
from jax.experimental import pallas as pl
from jax.experimental.pallas import tpu as pltpu
from jax.experimental.pallas import tpu_sc as plsc

_CUTOFF = 5.0
_NUM_BASIS = 16
_BR = 4000

_NC = 2
_NS = 16
_NW = _NC * _NS


def _bessel_body(e_ref, o_ref):
    e = e_ref[...]
    et = jnp.transpose(e)
    s = et * et
    r2 = jnp.sum(s, axis=0, keepdims=True)
    r = jnp.sqrt(r2)
    n = (lax.broadcasted_iota(jnp.int32, (_NUM_BASIS, r.shape[1]), 0) + 1
         ).astype(jnp.float32)
    y = jnp.sqrt(2.0 / _CUTOFF) * jnp.sin((jnp.pi / _CUTOFF) * n * r) / r
    o_ref[...] = jnp.transpose(y)


def _bessel_edges(edge_attr):
    ne = edge_attr.shape[0]
    grid = ne // _BR
    return pl.pallas_call(
        _bessel_body,
        grid=(grid,),
        in_specs=[pl.BlockSpec((_BR, 3), lambda i: (i, 0))],
        out_specs=pl.BlockSpec((_BR, _NUM_BASIS), lambda i: (i, 0)),
        out_shape=jax.ShapeDtypeStruct((ne, _NUM_BASIS), jnp.float32),
    )(edge_attr)


def _make_sc_gather(n, d):
    b_full = -(-n // _NW)
    b_full = ((b_full + 7) // 8) * 8
    b_last = n - (_NW - 1) * b_full
    assert 0 < b_last <= b_full and b_last % 8 == 0
    mesh = plsc.VectorSubcoreMesh(core_axis_name="c", subcore_axis_name="s")

    @functools.partial(
        pl.kernel,
        mesh=mesh,
        compiler_params=pltpu.CompilerParams(use_tc_tiling_on_sc=False),
        out_type=(jax.ShapeDtypeStruct((n, d), jnp.float32),
                  jax.ShapeDtypeStruct((n, d), jnp.float32)),
        scratch_types=[
            pltpu.VMEM((b_full,), jnp.int32),
            pltpu.VMEM((b_full, d), jnp.float32),
            pltpu.VMEM((b_full, d), jnp.float32),
            pltpu.SemaphoreType.DMA,
        ],
    )
    def gather(wx_hbm, wz_hbm, idx_hbm, outx_hbm, outz_hbm,
               idx_v, rx_v, rz_v, sem):
        wid = lax.axis_index("s") * _NC + lax.axis_index("c")
        base = wid * b_full

        @pl.when(wid < _NW - 1)
        def _full():
            pltpu.sync_copy(idx_hbm.at[pl.ds(base, b_full)], idx_v)
            pltpu.async_copy(wx_hbm.at[idx_v], rx_v, sem).wait()
            pltpu.async_copy(wz_hbm.at[idx_v], rz_v, sem).wait()
            pltpu.sync_copy(rx_v, outx_hbm.at[pl.ds(base, b_full)])
            pltpu.sync_copy(rz_v, outz_hbm.at[pl.ds(base, b_full)])

        @pl.when(wid == _NW - 1)
        def _last():
            idx_t = idx_v.at[pl.ds(0, b_last)]
            pltpu.sync_copy(idx_hbm.at[pl.ds(base, b_last)], idx_t)
            pltpu.async_copy(wx_hbm.at[idx_t], rx_v.at[pl.ds(0, b_last)],
                             sem).wait()
            pltpu.async_copy(wz_hbm.at[idx_t], rz_v.at[pl.ds(0, b_last)],
                             sem).wait()
            pltpu.sync_copy(rx_v.at[pl.ds(0, b_last)],
                            outx_hbm.at[pl.ds(base, b_last)])
            pltpu.sync_copy(rz_v.at[pl.ds(0, b_last)],
                            outz_hbm.at[pl.ds(base, b_last)])

    return gather


def kernel(x, edge_attr, W_x, W_z):
    n = x.shape[0]
    h_node_x, h_node_z = _make_sc_gather(n, W_x.shape[1])(W_x, W_z, x)
    h_edge = _bessel_edges(edge_attr)
    return h_node_x, h_node_z, h_edge


# speedup vs baseline: 46.0256x; 12.3910x over previous
import functools

import jax
import jax.numpy as jnp
from jax import lax
from jax.experimental import pallas as pl
from jax.experimental.pallas import tpu as pltpu
from jax.experimental.pallas import tpu_sc as plsc

_CUTOFF = 5.0
_NUM_BASIS = 16
_BC = 12800

_NC = 2
_NS = 16
_NW = _NC * _NS

_PI = 3.14159265358979
_S0 = 9.99999983e-01
_S1 = -1.66666515e-01
_S2 = 8.33296391e-03
_S3 = -1.98047481e-04
_S4 = 2.59809511e-06


def _bessel_body(e_ref, o_ref):
    e = e_ref[...]
    r2 = jnp.sum(e * e, axis=0, keepdims=True)
    r = jnp.sqrt(r2)
    scale = jnp.sqrt(2.0 / _CUTOFF) / r
    theta = (_PI / _CUTOFF) * r
    n = (lax.broadcasted_iota(jnp.int32, (_NUM_BASIS, e.shape[1]), 0) + 1
         ).astype(jnp.float32)
    t = n * theta
    k = jnp.round(t * (1.0 / _PI))
    u = t - k * _PI
    s = u * u
    p = _S4
    for c in (_S3, _S2, _S1, _S0):
        p = p * s + c
    p = p * u
    odd = (k.astype(jnp.int32) & 1) == 1
    o_ref[...] = jnp.where(odd, -p, p) * scale


def _bessel_edges_t(ea_t):
    ne = ea_t.shape[1]
    grid = ne // _BC
    assert grid * _BC == ne
    return pl.pallas_call(
        _bessel_body,
        grid=(grid,),
        in_specs=[pl.BlockSpec((3, _BC), lambda i: (0, i))],
        out_specs=pl.BlockSpec((_NUM_BASIS, _BC), lambda i: (0, i)),
        out_shape=jax.ShapeDtypeStruct((_NUM_BASIS, ne), jnp.float32),
    )(ea_t)


def _make_sc_gather(n, d, species):
    n_pad = ((n + 127) // 128) * 128
    nw = _NS
    b_full = ((-(-n_pad // nw) + 127) // 128) * 128
    b_last = n_pad - (nw - 1) * b_full
    r_last = n - (nw - 1) * b_full
    assert 0 < b_last <= b_full and b_last % 128 == 0
    assert 0 < r_last <= b_last and r_last % 8 == 0 and (b_last - r_last) % 16 == 0
    mesh = plsc.VectorSubcoreMesh(core_axis_name="c", subcore_axis_name="s",
                                  num_cores=1)

    @functools.partial(
        pl.kernel,
        mesh=mesh,
        compiler_params=pltpu.CompilerParams(use_tc_tiling_on_sc=True,
                                            needs_layout_passes=False),
        out_type=(jax.ShapeDtypeStruct((d, n_pad), jnp.float32),
                  jax.ShapeDtypeStruct((d, n_pad), jnp.float32)),
        scratch_types=[
            pltpu.VMEM((d, species), jnp.float32),
            pltpu.VMEM((d, species), jnp.float32),
            pltpu.VMEM((b_full,), jnp.int32),
            pltpu.VMEM((d, b_full), jnp.float32),
        ],
    )
    def gather(wxt_hbm, wzt_hbm, idx_hbm, ox_hbm, oz_hbm,
               wx_v, wz_v, idx_v, out_v):
        wid = lax.axis_index("s")
        base = wid * b_full
        pltpu.sync_copy(wxt_hbm, wx_v)
        pltpu.sync_copy(wzt_hbm, wz_v)

        def run(n_idx, size):
            pltpu.sync_copy(idx_hbm.at[pl.ds(base, n_idx)],
                            idx_v.at[pl.ds(0, n_idx)])
            if n_idx < size:
                zeros = jnp.zeros((16,), jnp.int32)
                for off in range(n_idx, size, 16):
                    idx_v[pl.ds(off, 16)] = zeros

            def table_pass(w_v, o_hbm):
                def body(ci, _):
                    off = ci * 16
                    idx = idx_v[pl.ds(off, 16)]
                    for f in range(d):
                        fvec = jnp.full((16,), f, jnp.int32)
                        vals = plsc.load_gather(w_v, [fvec, idx])
                        out_v[f, pl.ds(off, 16)] = vals
                    return 0

                lax.fori_loop(0, size // 16, body, 0)
                pltpu.sync_copy(out_v.at[:, pl.ds(0, size)],
                                o_hbm.at[:, pl.ds(base, size)])

            table_pass(wx_v, ox_hbm)
            table_pass(wz_v, oz_hbm)

        @pl.when(wid < nw - 1)
        def _full():
            run(b_full, b_full)

        @pl.when(wid == nw - 1)
        def _last():
            run(r_last, b_last)

    return gather


def kernel(x, edge_attr, W_x, W_z):
    n = x.shape[0]
    d = W_x.shape[1]
    gx, gz = _make_sc_gather(n, d, W_x.shape[0])(W_x.T, W_z.T, x)
    he_t = _bessel_edges_t(edge_attr.T)
    return gx[:, :n].T, gz[:, :n].T, he_t.T


# speedup vs baseline: 69.3468x; 1.5067x over previous
import functools

import jax
import jax.numpy as jnp
from jax import lax
from jax.experimental import pallas as pl
from jax.experimental.pallas import tpu as pltpu
from jax.experimental.pallas import tpu_sc as plsc

_CUTOFF = 5.0
_NUM_BASIS = 16
_BC = 32000

_NC = 2
_NS = 16
_NW = _NC * _NS

_PI = 3.14159265358979
_INV_C = 1.0 / _CUTOFF
_SQ2C = 0.6324555320336759
_P0 = 9.99997486e-01
_P1 = -1.66651677e-01
_P2 = 8.30951228e-03
_P3 = -1.84470858e-04


def _bessel_body(e_ref, o_ref):
    e = e_ref[...]
    r2 = jnp.sum(e * e, axis=0, keepdims=True)
    irs = lax.rsqrt(r2)
    scale = _SQ2C * irs
    thpi = _INV_C * (r2 * irs)
    n = (lax.broadcasted_iota(jnp.int32, (_NUM_BASIS, e.shape[1]), 0) + 1
         ).astype(jnp.float32)
    m = n * thpi
    k = jnp.round(m)
    u = (m - k) * _PI
    s = u * u
    p = _P3
    for c in (_P2, _P1, _P0):
        p = p * s + c
    p = p * u
    sb = k.astype(jnp.int32) << 31
    y = lax.bitcast_convert_type(
        lax.bitcast_convert_type(p, jnp.int32) ^ sb, jnp.float32)
    o_ref[...] = y * scale


def _bessel_edges_t(ea_t):
    ne = ea_t.shape[1]
    grid = ne // _BC
    assert grid * _BC == ne
    return pl.pallas_call(
        _bessel_body,
        grid=(grid,),
        in_specs=[pl.BlockSpec((3, _BC), lambda i: (0, i))],
        out_specs=pl.BlockSpec((_NUM_BASIS, _BC), lambda i: (0, i)),
        out_shape=jax.ShapeDtypeStruct((_NUM_BASIS, ne), jnp.float32),
    )(ea_t)


def _make_sc_gather(n, d, species):
    n_pad = ((n + 127) // 128) * 128
    nw = _NS
    b_full = ((-(-n_pad // nw) + 127) // 128) * 128
    b_last = n_pad - (nw - 1) * b_full
    r_last = n - (nw - 1) * b_full
    assert 0 < b_last <= b_full and b_last % 128 == 0
    assert 0 < r_last <= b_last and r_last % 8 == 0 and (b_last - r_last) % 16 == 0
    mesh = plsc.VectorSubcoreMesh(core_axis_name="c", subcore_axis_name="s",
                                  num_cores=1)

    @functools.partial(
        pl.kernel,
        mesh=mesh,
        compiler_params=pltpu.CompilerParams(use_tc_tiling_on_sc=True,
                                            needs_layout_passes=False),
        out_type=(jax.ShapeDtypeStruct((d, n_pad), jnp.float32),
                  jax.ShapeDtypeStruct((d, n_pad), jnp.float32)),
        scratch_types=[
            pltpu.VMEM((d, species), jnp.float32),
            pltpu.VMEM((d, species), jnp.float32),
            pltpu.VMEM((b_full,), jnp.int32),
            pltpu.VMEM((d, b_full), jnp.float32),
        ],
    )
    def gather(wxt_hbm, wzt_hbm, idx_hbm, ox_hbm, oz_hbm,
               wx_v, wz_v, idx_v, out_v):
        wid = lax.axis_index("s")
        base = wid * b_full
        pltpu.sync_copy(wxt_hbm, wx_v)
        pltpu.sync_copy(wzt_hbm, wz_v)

        def run(n_idx, size):
            pltpu.sync_copy(idx_hbm.at[pl.ds(base, n_idx)],
                            idx_v.at[pl.ds(0, n_idx)])
            if n_idx < size:
                zeros = jnp.zeros((16,), jnp.int32)
                for off in range(n_idx, size, 16):
                    idx_v[pl.ds(off, 16)] = zeros

            def table_pass(w_v, o_hbm):
                def body(ci, _):
                    off = ci * 16
                    idx = idx_v[pl.ds(off, 16)]
                    for f in range(d):
                        fvec = jnp.full((16,), f, jnp.int32)
                        vals = plsc.load_gather(w_v, [fvec, idx])
                        out_v[f, pl.ds(off, 16)] = vals
                    return 0

                lax.fori_loop(0, size // 16, body, 0)
                pltpu.sync_copy(out_v.at[:, pl.ds(0, size)],
                                o_hbm.at[:, pl.ds(base, size)])

            table_pass(wx_v, ox_hbm)
            table_pass(wz_v, oz_hbm)

        @pl.when(wid < nw - 1)
        def _full():
            run(b_full, b_full)

        @pl.when(wid == nw - 1)
        def _last():
            run(r_last, b_last)

    return gather


def kernel(x, edge_attr, W_x, W_z):
    n = x.shape[0]
    d = W_x.shape[1]
    gx, gz = _make_sc_gather(n, d, W_x.shape[0])(W_x.T, W_z.T, x)
    he_t = _bessel_edges_t(edge_attr.T)
    return gx[:, :n].T, gz[:, :n].T, he_t.T
